# baseline (speedup 1.0000x reference)
import functools

import jax
import jax.numpy as jnp
from jax.experimental import pallas as pl
from jax.experimental.pallas import tpu as pltpu

_K = 5
_PAD = (_K - 1) // 2


def _body(x_ref, w1l_ref, w1r_ref, w2_ref, b1l_ref, b1r_ref, b2_ref,
          mw_ref, mh_ref, o_ref, xp_ref, sl_ref, sr_ref, *, C, OC, H, W):
    Wp = W + 2 * _PAD
    Hp = H + 2 * _PAD
    HWp = H * Wp
    HpWp = Hp * Wp
    f16 = jnp.bfloat16

    xp_ref[...] = jnp.zeros_like(xp_ref)
    xb = x_ref[0]
    for h in range(H):
        base = _PAD + (h + _PAD) * Wp + _PAD
        xp_ref[:, base:base + W] = xb[:, h * W:(h + 1) * W]

    bl = jnp.concatenate(
        [xp_ref[:, _PAD + t * Wp:_PAD + t * Wp + HWp] for t in range(_K)],
        axis=0)
    yl = jnp.dot(w1l_ref[...], bl, preferred_element_type=jnp.float32)
    yl = yl + b1l_ref[...] * mw_ref[...]
    sl_ref[:, :_PAD] = jnp.zeros((OC, _PAD), f16)
    sl_ref[:, _PAD:_PAD + HWp] = yl.astype(f16)
    sl_ref[:, _PAD + HWp:] = jnp.zeros((OC, _PAD), f16)

    br = jnp.concatenate([xp_ref[:, t:t + HpWp] for t in range(_K)], axis=0)
    yr = jnp.dot(w1r_ref[...], br, preferred_element_type=jnp.float32)
    yr = yr + b1r_ref[...] * mh_ref[...]
    sr_ref[...] = yr.astype(f16)

    b2 = jnp.concatenate(
        [sl_ref[:, t:t + HWp] for t in range(_K)]
        + [sr_ref[:, t * Wp:t * Wp + HWp] for t in range(_K)], axis=0)
    acc = jnp.dot(w2_ref[...], b2, preferred_element_type=jnp.float32)
    acc = acc + b2_ref[...]
    res = jnp.where(acc >= 0.0, acc, 0.1 * acc)

    for h in range(H):
        o_ref[0, :, h * W:(h + 1) * W] = res[:, h * Wp + _PAD:h * Wp + _PAD + W]


@jax.jit
def _forward(x, wl1, bl1, wl2, bl2, wr1, br1, wr2, br2):
    N, C, H, W = x.shape
    OC = wl1.shape[0]
    Hp, Wp = H + 2 * _PAD, W + 2 * _PAD
    HWp, HpWp = H * Wp, Hp * Wp
    XF = HpWp + 2 * _PAD

    f16 = jnp.bfloat16
    w1l = jnp.transpose(wl1[:, :, :, 0], (0, 2, 1)).reshape(OC, _K * C).astype(f16)
    w1r = jnp.transpose(wr1[:, :, 0, :], (0, 2, 1)).reshape(OC, _K * C).astype(f16)
    w2l = jnp.transpose(wl2[:, :, 0, :], (0, 2, 1)).reshape(OC, _K * OC).astype(f16)
    w2r = jnp.transpose(wr2[:, :, :, 0], (0, 2, 1)).reshape(OC, _K * OC).astype(f16)
    w2 = jnp.concatenate([w2l, w2r], axis=1)
    b1l = bl1[:, None]
    b1r = br1[:, None]
    b2 = (bl2 + br2)[:, None]

    colw = jnp.arange(HWp, dtype=jnp.int32) % Wp
    mw = ((colw >= _PAD) & (colw < _PAD + W)).astype(jnp.float32)[None, :]
    rowh = jnp.arange(HpWp, dtype=jnp.int32) // Wp
    mh = ((rowh >= _PAD) & (rowh < _PAD + H)).astype(jnp.float32)[None, :]

    out = pl.pallas_call(
        functools.partial(_body, C=C, OC=OC, H=H, W=W),
        out_shape=jax.ShapeDtypeStruct((N, OC, H * W), jnp.float32),
        grid=(N,),
        in_specs=[
            pl.BlockSpec((1, C, H * W), lambda n: (n, 0, 0)),
            pl.BlockSpec((OC, _K * C), lambda n: (0, 0)),
            pl.BlockSpec((OC, _K * C), lambda n: (0, 0)),
            pl.BlockSpec((OC, 2 * _K * OC), lambda n: (0, 0)),
            pl.BlockSpec((OC, 1), lambda n: (0, 0)),
            pl.BlockSpec((OC, 1), lambda n: (0, 0)),
            pl.BlockSpec((OC, 1), lambda n: (0, 0)),
            pl.BlockSpec((1, HWp), lambda n: (0, 0)),
            pl.BlockSpec((1, HpWp), lambda n: (0, 0)),
        ],
        out_specs=pl.BlockSpec((1, OC, H * W), lambda n: (n, 0, 0)),
        scratch_shapes=[
            pltpu.VMEM((C, XF), f16),
            pltpu.VMEM((OC, HWp + 2 * _PAD), f16),
            pltpu.VMEM((OC, HpWp), f16),
        ],
        compiler_params=pltpu.CompilerParams(
            dimension_semantics=("parallel",),
            vmem_limit_bytes=48 * 1024 * 1024,
        ),
    )(x.reshape(N, C, H * W).astype(f16), w1l, w1r, w2, b1l, b1r, b2, mw, mh)

    return out.reshape(N, OC, H, W)


def kernel(x, wl1, bl1, wl2, bl2, wr1, br1, wr2, br2):
    return _forward(x, wl1, bl1, wl2, bl2, wr1, br1, wr2, br2)


# speedup vs baseline: 1.0660x; 1.0660x over previous
import functools

import jax
import jax.numpy as jnp
from jax.experimental import pallas as pl
from jax.experimental.pallas import tpu as pltpu

_K = 5
_PAD = (_K - 1) // 2


def _body(x_ref, w1l_ref, w1r_ref, w2_ref, b1l_ref, b1r_ref, b2_ref,
          mw_ref, mh_ref, o_ref, xp_ref, sl_ref, sr_ref, *, C, OC, H, W):
    Wp = W + 2 * _PAD
    Hp = H + 2 * _PAD
    HWp = H * Wp
    HpWp = Hp * Wp
    f16 = jnp.bfloat16

    for i in range(2):
        xp_ref[i] = jnp.zeros_like(xp_ref[i])
        xb = x_ref[i]
        for h in range(H):
            base = _PAD + (h + _PAD) * Wp + _PAD
            xp_ref[i, :, base:base + W] = xb[:, h * W:(h + 1) * W].astype(f16)

        bl = jnp.concatenate(
            [xp_ref[i, :, _PAD + t * Wp:_PAD + t * Wp + HWp]
             for t in range(_K)], axis=0)
        yl = jnp.dot(w1l_ref[...], bl, preferred_element_type=jnp.float32)
        yl = yl + b1l_ref[...] * mw_ref[...]
        sl_ref[:, :_PAD] = jnp.zeros((OC, _PAD), f16)
        sl_ref[:, _PAD:_PAD + HWp] = yl.astype(f16)
        sl_ref[:, _PAD + HWp:] = jnp.zeros((OC, _PAD), f16)

        br = jnp.concatenate(
            [xp_ref[i, :, t:t + HpWp] for t in range(_K)], axis=0)
        yr = jnp.dot(w1r_ref[...], br, preferred_element_type=jnp.float32)
        yr = yr + b1r_ref[...] * mh_ref[...]
        sr_ref[...] = yr.astype(f16)

        b2 = jnp.concatenate(
            [sl_ref[:, t:t + HWp] for t in range(_K)]
            + [sr_ref[:, t * Wp:t * Wp + HWp] for t in range(_K)], axis=0)
        acc = jnp.dot(w2_ref[...], b2, preferred_element_type=jnp.float32)
        acc = acc + b2_ref[...]
        res = jnp.where(acc >= 0.0, acc, 0.1 * acc)

        for h in range(H):
            o_ref[i, :, h * W:(h + 1) * W] = (
                res[:, h * Wp + _PAD:h * Wp + _PAD + W])


@jax.jit
def _forward(x, wl1, bl1, wl2, bl2, wr1, br1, wr2, br2):
    N, C, H, W = x.shape
    OC = wl1.shape[0]
    Hp, Wp = H + 2 * _PAD, W + 2 * _PAD
    HWp, HpWp = H * Wp, Hp * Wp
    XF = HpWp + 2 * _PAD

    f16 = jnp.bfloat16
    w1l = jnp.transpose(wl1[:, :, :, 0], (0, 2, 1)).reshape(OC, _K * C).astype(f16)
    w1r = jnp.transpose(wr1[:, :, 0, :], (0, 2, 1)).reshape(OC, _K * C).astype(f16)
    w2l = jnp.transpose(wl2[:, :, 0, :], (0, 2, 1)).reshape(OC, _K * OC).astype(f16)
    w2r = jnp.transpose(wr2[:, :, :, 0], (0, 2, 1)).reshape(OC, _K * OC).astype(f16)
    w2 = jnp.concatenate([w2l, w2r], axis=1)
    b1l = bl1[:, None]
    b1r = br1[:, None]
    b2 = (bl2 + br2)[:, None]

    colw = jnp.arange(HWp, dtype=jnp.int32) % Wp
    mw = ((colw >= _PAD) & (colw < _PAD + W)).astype(jnp.float32)[None, :]
    rowh = jnp.arange(HpWp, dtype=jnp.int32) // Wp
    mh = ((rowh >= _PAD) & (rowh < _PAD + H)).astype(jnp.float32)[None, :]

    out = pl.pallas_call(
        functools.partial(_body, C=C, OC=OC, H=H, W=W),
        out_shape=jax.ShapeDtypeStruct((N, OC, H * W), jnp.float32),
        grid=(N // 2,),
        in_specs=[
            pl.BlockSpec((2, C, H * W), lambda n: (n, 0, 0)),
            pl.BlockSpec((OC, _K * C), lambda n: (0, 0)),
            pl.BlockSpec((OC, _K * C), lambda n: (0, 0)),
            pl.BlockSpec((OC, 2 * _K * OC), lambda n: (0, 0)),
            pl.BlockSpec((OC, 1), lambda n: (0, 0)),
            pl.BlockSpec((OC, 1), lambda n: (0, 0)),
            pl.BlockSpec((OC, 1), lambda n: (0, 0)),
            pl.BlockSpec((1, HWp), lambda n: (0, 0)),
            pl.BlockSpec((1, HpWp), lambda n: (0, 0)),
        ],
        out_specs=pl.BlockSpec((2, OC, H * W), lambda n: (n, 0, 0)),
        scratch_shapes=[
            pltpu.VMEM((2, C, XF), f16),
            pltpu.VMEM((OC, HWp + 2 * _PAD), f16),
            pltpu.VMEM((OC, HpWp), f16),
        ],
        compiler_params=pltpu.CompilerParams(
            dimension_semantics=("parallel",),
            vmem_limit_bytes=48 * 1024 * 1024,
        ),
    )(x.reshape(N, C, H * W), w1l, w1r, w2, b1l, b1r, b2, mw, mh)

    return out.reshape(N, OC, H, W)


def kernel(x, wl1, bl1, wl2, bl2, wr1, br1, wr2, br2):
    return _forward(x, wl1, bl1, wl2, bl2, wr1, br1, wr2, br2)


# speedup vs baseline: 1.1065x; 1.0380x over previous
import functools

import jax
import jax.numpy as jnp
from jax.experimental import pallas as pl
from jax.experimental.pallas import tpu as pltpu

_K = 5
_PAD = (_K - 1) // 2


def _body(x_ref, w1l_ref, w1r_ref, w2_ref, b1l_ref, b1r_ref, b2_ref,
          mw_ref, mh_ref, o_ref, xp_ref, sl_ref, sr_ref, *, C, OC, H, W):
    Wp = W + 2 * _PAD
    Hp = H + 2 * _PAD
    HWp = H * Wp
    HpWp = Hp * Wp
    f16 = jnp.bfloat16

    xp_ref[...] = jnp.zeros_like(xp_ref)
    xb = x_ref[0]
    for h in range(H):
        base = _PAD + (h + _PAD) * Wp + _PAD
        xp_ref[:, base:base + W] = xb[:, h * W:(h + 1) * W].astype(f16)

    bl = jnp.concatenate(
        [xp_ref[:, _PAD + t * Wp:_PAD + t * Wp + HWp] for t in range(_K)],
        axis=0)
    yl = jnp.dot(w1l_ref[...], bl, preferred_element_type=jnp.float32)

    br = jnp.concatenate([xp_ref[:, t:t + HpWp] for t in range(_K)], axis=0)

    yl = yl + b1l_ref[...] * mw_ref[...]
    sl_ref[:, :_PAD] = jnp.zeros((OC, _PAD), f16)
    sl_ref[:, _PAD:_PAD + HWp] = yl.astype(f16)
    sl_ref[:, _PAD + HWp:] = jnp.zeros((OC, _PAD), f16)

    yr = jnp.dot(w1r_ref[...], br, preferred_element_type=jnp.float32)

    b2l = jnp.concatenate([sl_ref[:, t:t + HWp] for t in range(_K)], axis=0)

    yr = yr + b1r_ref[...] * mh_ref[...]
    sr_ref[...] = yr.astype(f16)

    b2 = jnp.concatenate(
        [b2l] + [sr_ref[:, t * Wp:t * Wp + HWp] for t in range(_K)], axis=0)
    acc = jnp.dot(w2_ref[...], b2, preferred_element_type=jnp.float32)
    acc = acc + b2_ref[...]
    res = jnp.where(acc >= 0.0, acc, 0.1 * acc)

    for h in range(H):
        o_ref[0, :, h * W:(h + 1) * W] = res[:, h * Wp + _PAD:h * Wp + _PAD + W]


@jax.jit
def _forward(x, wl1, bl1, wl2, bl2, wr1, br1, wr2, br2):
    N, C, H, W = x.shape
    OC = wl1.shape[0]
    Hp, Wp = H + 2 * _PAD, W + 2 * _PAD
    HWp, HpWp = H * Wp, Hp * Wp
    XF = HpWp + 2 * _PAD

    f16 = jnp.bfloat16
    w1l = jnp.transpose(wl1[:, :, :, 0], (0, 2, 1)).reshape(OC, _K * C).astype(f16)
    w1r = jnp.transpose(wr1[:, :, 0, :], (0, 2, 1)).reshape(OC, _K * C).astype(f16)
    w2l = jnp.transpose(wl2[:, :, 0, :], (0, 2, 1)).reshape(OC, _K * OC).astype(f16)
    w2r = jnp.transpose(wr2[:, :, :, 0], (0, 2, 1)).reshape(OC, _K * OC).astype(f16)
    w2 = jnp.concatenate([w2l, w2r], axis=1)
    b1l = bl1[:, None]
    b1r = br1[:, None]
    b2 = (bl2 + br2)[:, None]

    colw = jnp.arange(HWp, dtype=jnp.int32) % Wp
    mw = ((colw >= _PAD) & (colw < _PAD + W)).astype(jnp.float32)[None, :]
    rowh = jnp.arange(HpWp, dtype=jnp.int32) // Wp
    mh = ((rowh >= _PAD) & (rowh < _PAD + H)).astype(jnp.float32)[None, :]

    out = pl.pallas_call(
        functools.partial(_body, C=C, OC=OC, H=H, W=W),
        out_shape=jax.ShapeDtypeStruct((N, OC, H * W), jnp.float32),
        grid=(N,),
        in_specs=[
            pl.BlockSpec((1, C, H * W), lambda n: (n, 0, 0)),
            pl.BlockSpec((OC, _K * C), lambda n: (0, 0)),
            pl.BlockSpec((OC, _K * C), lambda n: (0, 0)),
            pl.BlockSpec((OC, 2 * _K * OC), lambda n: (0, 0)),
            pl.BlockSpec((OC, 1), lambda n: (0, 0)),
            pl.BlockSpec((OC, 1), lambda n: (0, 0)),
            pl.BlockSpec((OC, 1), lambda n: (0, 0)),
            pl.BlockSpec((1, HWp), lambda n: (0, 0)),
            pl.BlockSpec((1, HpWp), lambda n: (0, 0)),
        ],
        out_specs=pl.BlockSpec((1, OC, H * W), lambda n: (n, 0, 0)),
        scratch_shapes=[
            pltpu.VMEM((C, XF), f16),
            pltpu.VMEM((OC, HWp + 2 * _PAD), f16),
            pltpu.VMEM((OC, HpWp), f16),
        ],
        compiler_params=pltpu.CompilerParams(
            dimension_semantics=("parallel",),
            vmem_limit_bytes=48 * 1024 * 1024,
        ),
    )(x.reshape(N, C, H * W), w1l, w1r, w2, b1l, b1r, b2, mw, mh)

    return out.reshape(N, OC, H, W)


def kernel(x, wl1, bl1, wl2, bl2, wr1, br1, wr2, br2):
    return _forward(x, wl1, bl1, wl2, bl2, wr1, br1, wr2, br2)
